# Initial kernel scaffold; baseline (speedup 1.0000x reference)
import jax
import jax.numpy as jnp
from jax.experimental import pallas as pl


def kernel(word_inputs, char_ids, char_lengths, word_table, char_table, W, b):
    raise NotImplementedError("write your pallas kernel here")


# speedup vs baseline: 18.8063x; 18.8063x over previous
import functools

import jax
import jax.numpy as jnp
from jax import lax
from jax.experimental import pallas as pl
from jax.experimental.pallas import tpu as pltpu
from jax.experimental.pallas import tpu_sc as plsc

B, S, MAXC = 1024, 200, 16
DW, DC, DOUT = 128, 64, 128
N = B * S
NC, NS = 2, 16
NW = NC * NS
TPW = N // NW
C = 128
NCHUNK = TPW // C
VEC = 16
NVEC = DOUT // VEC


def _proj_body(t_ref, w_ref, b_ref, out_ref):
    out_ref[...] = (
        jnp.dot(t_ref[...], w_ref[...], preferred_element_type=jnp.float32)
        + b_ref[...]
    )


def _project(table, w, b2d, bm):
    m, k = table.shape
    return pl.pallas_call(
        _proj_body,
        grid=(m // bm,),
        in_specs=[
            pl.BlockSpec((bm, k), lambda i: (i, 0)),
            pl.BlockSpec((k, DOUT), lambda i: (0, 0)),
            pl.BlockSpec((1, DOUT), lambda i: (0, 0)),
        ],
        out_specs=pl.BlockSpec((bm, DOUT), lambda i: (i, 0)),
        out_shape=jax.ShapeDtypeStruct((m, DOUT), jnp.float32),
    )(table, w, b2d)


@functools.partial(
    pl.kernel,
    out_type=jax.ShapeDtypeStruct((N, DOUT), jnp.float32),
    mesh=plsc.VectorSubcoreMesh(core_axis_name="c", subcore_axis_name="s"),
    scratch_types=[
        pltpu.VMEM((256, DOUT), jnp.float32),
        pltpu.VMEM((C,), jnp.int32),
        pltpu.VMEM((C, MAXC), jnp.int32),
        pltpu.VMEM((C,), jnp.int32),
        pltpu.VMEM((C, DOUT), jnp.float32),
        pltpu.SemaphoreType.DMA,
    ],
    compiler_params=pltpu.CompilerParams(needs_layout_passes=False),
)
def _sc_embed(pw_hbm, pc_hbm, widx_hbm, cid_hbm, len_hbm, out_hbm,
              pc_v, idx_v, cid_v, len_v, rows_v, sem):
    wid = lax.axis_index("s") * NC + lax.axis_index("c")
    pltpu.sync_copy(pc_hbm, pc_v)
    lane = lax.iota(jnp.int32, VEC)
    offs = [lane + VEC * c for c in range(NVEC)]

    def chunk_body(g, carry):
        base = wid * TPW + g * C
        pltpu.sync_copy(widx_hbm.at[pl.ds(base, C)], idx_v)
        pltpu.sync_copy(cid_hbm.at[pl.ds(base, C)], cid_v)
        pltpu.sync_copy(len_hbm.at[pl.ds(base, C)], len_v)
        pltpu.async_copy(pw_hbm.at[idx_v], rows_v, sem).wait()

        def grp_body(tg, carry2):
            t0 = tg * VEC
            lens = len_v[pl.ds(t0, VEC)]
            for k in range(VEC):
                t = t0 + k
                nchars = lens[k]
                cvec = cid_v[t, :]
                accs = tuple(rows_v[t, pl.ds(VEC * c, VEC)] for c in range(NVEC))

                def char_body(j, a):
                    rv = cvec.at[jnp.full((VEC,), 0, jnp.int32) + j].get(
                        mode="promise_in_bounds")
                    return tuple(
                        a[c] + plsc.load_gather(pc_v, [rv, offs[c]])
                        for c in range(NVEC)
                    )

                accs = lax.fori_loop(0, nchars, char_body, accs)
                for c in range(NVEC):
                    rows_v[t, pl.ds(VEC * c, VEC)] = accs[c]
            return carry2

        lax.fori_loop(0, C // VEC, grp_body, 0)
        pltpu.sync_copy(rows_v, out_hbm.at[pl.ds(base, C)])
        return carry

    lax.fori_loop(0, NCHUNK, chunk_body, 0)


def kernel(word_inputs, char_ids, char_lengths, word_table, char_table, W, b):
    ww = W[:DW]
    wc = W[DW:]
    pw = _project(word_table, ww, b.reshape(1, DOUT), 1000)
    pc = _project(char_table, wc, jnp.zeros((1, DOUT), jnp.float32), 256)
    widx = word_inputs.reshape(N).astype(jnp.int32)
    cid = char_ids.reshape(N, MAXC).astype(jnp.int32)
    clen = char_lengths.reshape(N).astype(jnp.int32)
    out = _sc_embed(pw, pc, widx, cid, clen)
    return out.reshape(B, S, DOUT)


# speedup vs baseline: 20.0094x; 1.0640x over previous
import functools

import jax
import jax.numpy as jnp
from jax import lax
from jax.experimental import pallas as pl
from jax.experimental.pallas import tpu as pltpu
from jax.experimental.pallas import tpu_sc as plsc

B, S, MAXC = 1024, 200, 16
DW, DC, DOUT = 128, 64, 128
N = B * S
NC, NS = 2, 16
NW = NC * NS
TPW = N // NW
C = 128
NCHUNK = TPW // C
VEC = 16
NGRP = DOUT // 32

_PERM = tuple(
    32 * (p // 32) + (16 if p % 2 else 0) + (p % 32) // 2 for p in range(DOUT)
)


def _proj_f32_body(t_ref, w_ref, b_ref, out_ref):
    out_ref[...] = (
        jnp.dot(t_ref[...], w_ref[...], preferred_element_type=jnp.float32)
        + b_ref[...]
    )


def _proj_bf16_body(t_ref, w_ref, b_ref, out_ref):
    acc = (
        jnp.dot(t_ref[...], w_ref[...], preferred_element_type=jnp.float32)
        + b_ref[...]
    )
    out_ref[...] = acc.astype(jnp.bfloat16)


def _project(table, w, b2d, bm, body, out_cols, out_dtype):
    m, k = table.shape
    return pl.pallas_call(
        body,
        grid=(m // bm,),
        in_specs=[
            pl.BlockSpec((bm, k), lambda i: (i, 0)),
            pl.BlockSpec((k, DOUT), lambda i: (0, 0)),
            pl.BlockSpec((1, DOUT), lambda i: (0, 0)),
        ],
        out_specs=pl.BlockSpec((bm, out_cols), lambda i: (i, 0)),
        out_shape=jax.ShapeDtypeStruct((m, out_cols), out_dtype),
    )(table, w, b2d)


@functools.partial(
    pl.kernel,
    out_type=jax.ShapeDtypeStruct((N, DOUT), jnp.float32),
    mesh=plsc.VectorSubcoreMesh(core_axis_name="c", subcore_axis_name="s"),
    scratch_types=[
        pltpu.VMEM((256, DOUT // 2), jnp.int32),
        pltpu.VMEM((C,), jnp.int32),
        pltpu.VMEM((C, MAXC), jnp.int32),
        pltpu.VMEM((C,), jnp.int32),
        pltpu.VMEM((C, DOUT), jnp.float32),
        pltpu.VMEM((C, DOUT), jnp.float32),
        pltpu.SemaphoreType.DMA,
    ],
    compiler_params=pltpu.CompilerParams(needs_layout_passes=False),
)
def _sc_embed(pw_hbm, pc_hbm, widx_hbm, cid_hbm, len_hbm, out_hbm,
              pc_v, idx_v, cid_v, len_v, rows_v, orow_v, sem):
    wid = lax.axis_index("s") * NC + lax.axis_index("c")
    pltpu.sync_copy(pc_hbm, pc_v)
    lane = lax.iota(jnp.int32, VEC)
    offs = [lane + VEC * c for c in range(NGRP)]

    def chunk_body(g, carry):
        base = wid * TPW + g * C
        pltpu.sync_copy(widx_hbm.at[pl.ds(base, C)], idx_v)
        pltpu.sync_copy(cid_hbm.at[pl.ds(base, C)], cid_v)
        pltpu.sync_copy(len_hbm.at[pl.ds(base, C)], len_v)
        pltpu.async_copy(pw_hbm.at[idx_v], rows_v, sem).wait()

        def grp_body(tg, carry2):
            t0 = tg * VEC
            lens = len_v[pl.ds(t0, VEC)]
            for k in range(VEC):
                t = t0 + k
                nchars = lens[k]
                cvec = cid_v[t, :]
                accs = []
                for c in range(NGRP):
                    accs.append(rows_v[t, pl.ds(32 * c, VEC)])
                    accs.append(rows_v[t, pl.ds(32 * c + VEC, VEC)])
                accs = tuple(accs)

                def char_body(j, a):
                    rv = cvec.at[jnp.full((VEC,), 0, jnp.int32) + j].get(
                        mode="promise_in_bounds")
                    out = []
                    for c in range(NGRP):
                        gi = plsc.load_gather(pc_v, [rv, offs[c]])
                        gb = plsc.bitcast(gi, jnp.bfloat16)
                        da, db = plsc.unpack(
                            gb, format=plsc.PackFormat.INTERLEAVED)
                        out.append(a[2 * c] + da)
                        out.append(a[2 * c + 1] + db)
                    return tuple(out)

                accs = lax.fori_loop(0, nchars, char_body, accs)
                for c in range(NGRP):
                    orow_v[t, pl.ds(32 * c, VEC)] = accs[2 * c]
                    orow_v[t, pl.ds(32 * c + VEC, VEC)] = accs[2 * c + 1]
            return carry2

        lax.fori_loop(0, C // VEC, grp_body, 0)
        pltpu.sync_copy(orow_v, out_hbm.at[pl.ds(base, C)])
        return carry

    lax.fori_loop(0, NCHUNK, chunk_body, 0)


def kernel(word_inputs, char_ids, char_lengths, word_table, char_table, W, b):
    perm = jnp.array(_PERM, dtype=jnp.int32)
    wp = W[:, perm]
    pw = _project(word_table, W[:DW], b.reshape(1, DOUT), 1000,
                  _proj_f32_body, DOUT, jnp.float32)
    pc_bf = _project(char_table, wp[DW:], jnp.zeros((1, DOUT), jnp.float32),
                     256, _proj_bf16_body, DOUT, jnp.bfloat16)
    pc_i32 = lax.bitcast_convert_type(
        pc_bf.reshape(-1, DOUT // 2, 2), jnp.int32)
    widx = word_inputs.reshape(N).astype(jnp.int32)
    cid = char_ids.reshape(N, MAXC).astype(jnp.int32)
    clen = char_lengths.reshape(N).astype(jnp.int32)
    out = _sc_embed(pw, pc_i32, widx, cid, clen)
    return out.reshape(B, S, DOUT)
